# baseline (speedup 1.0000x reference)
import functools

import jax
import jax.numpy as jnp
import numpy as np
from jax import lax
from jax.experimental import pallas as pl
from jax.experimental.pallas import tpu as pltpu
from jax.experimental.pallas import tpu_sc as plsc

NC = 2
NS = 16
NW = NC * NS

CH = 80
LANES = 16
NORM = 100.0


def _silu(x):
    return x * jax.nn.sigmoid(x)


def _k1_body(h_ref, w1s_ref, w1t_ref, o1_ref, o2_ref):
    hb = h_ref[...]
    o1_ref[...] = jnp.dot(hb, w1s_ref[...], preferred_element_type=jnp.float32)
    o2_ref[...] = jnp.dot(hb, w1t_ref[...], preferred_element_type=jnp.float32)


def _k1(h, w1s, w1t, bn):
    n, nf = h.shape
    hid = w1s.shape[1]
    return pl.pallas_call(
        _k1_body,
        grid=(n // bn,),
        in_specs=[
            pl.BlockSpec((bn, nf), lambda i: (i, 0)),
            pl.BlockSpec((nf, hid), lambda i: (0, 0)),
            pl.BlockSpec((nf, hid), lambda i: (0, 0)),
        ],
        out_specs=[
            pl.BlockSpec((bn, hid), lambda i: (i, 0)),
            pl.BlockSpec((bn, hid), lambda i: (i, 0)),
        ],
        out_shape=[
            jax.ShapeDtypeStruct((n, hid), jnp.float32),
            jax.ShapeDtypeStruct((n, hid), jnp.float32),
        ],
    )(h, w1s, w1t)


def _sc_gather_sum(g1, g2, row1d, col1d, e, hid, beh):
    wrpt = e // (2 * NW)
    CW = 40
    nch = wrpt // CW
    G = 1000
    ng = wrpt // G
    assert nch % 2 == 1 and nch >= 3
    assert beh % G == 0 and wrpt % G == 0 and G % CW == 0
    npair = (nch - 1) // 2
    mesh = plsc.VectorSubcoreMesh(
        core_axis_name="c", subcore_axis_name="s",
        num_cores=NC, num_subcores=NS)

    def body(g1_hbm, g2_hbm, row_hbm, col_hbm, out_hbm,
             iar, iac, ibr, ibc,
             a10, a20, b10, b20, a11, a21, b11, b21,
             pb0, pb1, sg0, sg1, so0, so1, sp):
        cid = lax.axis_index("c")
        sid = lax.axis_index("s")
        wid = sid * NC + cid
        m0t = wid * wrpt

        cps = []
        for g in range(ng):
            gg = wid * ng + g
            bi = gg >> 1
            mm = (gg & 1) * G
            src_a = pl.multiple_of(2 * beh * bi + mm, 8)
            src_b = src_a + beh
            dst = pl.ds(g * G, G)
            cps.append(pltpu.async_copy(row_hbm.at[pl.ds(src_a, G)], iar.at[dst], sp))
            cps.append(pltpu.async_copy(col_hbm.at[pl.ds(src_a, G)], iac.at[dst], sp))
            cps.append(pltpu.async_copy(row_hbm.at[pl.ds(src_b, G)], ibr.at[dst], sp))
            cps.append(pltpu.async_copy(col_hbm.at[pl.ds(src_b, G)], ibc.at[dst], sp))
        for cp in cps:
            cp.wait()

        def fire(c, a1, a2, b1_, b2_, sg):
            sl = pl.ds(c * CW, CW)
            pltpu.async_copy(g1_hbm.at[iar.at[sl]], a1, sg)
            pltpu.async_copy(g2_hbm.at[iac.at[sl]], a2, sg)
            pltpu.async_copy(g1_hbm.at[ibr.at[sl]], b1_, sg)
            pltpu.async_copy(g2_hbm.at[ibc.at[sl]], b2_, sg)

        def wait_gather(c, a1, a2, b1_, b2_, sg):
            sl = pl.ds(c * CW, CW)
            pltpu.make_async_copy(g1_hbm.at[iar.at[sl]], a1, sg).wait()
            pltpu.make_async_copy(g2_hbm.at[iac.at[sl]], a2, sg).wait()
            pltpu.make_async_copy(g1_hbm.at[ibr.at[sl]], b1_, sg).wait()
            pltpu.make_async_copy(g2_hbm.at[ibc.at[sl]], b2_, sg).wait()

        def bf16_word(x1, x2, r, sl_lo, sl_hi):
            va = x1[r, sl_lo] + x2[r, sl_lo]
            vb = x1[r, sl_hi] + x2[r, sl_hi]
            ia = lax.bitcast_convert_type(va, jnp.int32)
            ib = lax.bitcast_convert_type(vb, jnp.int32)
            ra = ia + 0x7FFF + ((ia >> 16) & 1)
            rb = ib + 0x7FFF + ((ib >> 16) & 1)
            return ((ra >> 16) & 0xFFFF) | (rb & -65536)

        def pack(a1, a2, b1_, b2_, pb):
            hw = hid // 2

            def rowfn(rr, c2):
                for t in range(hw // LANES):
                    sl_lo = pl.ds(t * LANES, LANES)
                    sl_hi = pl.ds(hw + t * LANES, LANES)
                    pb[rr, sl_lo] = bf16_word(a1, a2, rr, sl_lo, sl_hi)
                    pb[rr, sl_hi] = bf16_word(b1_, b2_, rr, sl_lo, sl_hi)
                return c2

            lax.fori_loop(0, CW, rowfn, 0)

        def store(c, pb, so):
            pltpu.async_copy(pb, out_hbm.at[pl.ds(m0t + c * CW, CW)], so)

        def wait_store(c, pb, so):
            pltpu.make_async_copy(
                pb, out_hbm.at[pl.ds(m0t + c * CW, CW)], so).wait()

        fire(0, a10, a20, b10, b20, sg0)

        def pair(i2, carry):
            c0 = 2 * i2
            c1 = c0 + 1
            fire(c1, a11, a21, b11, b21, sg1)
            wait_gather(c0, a10, a20, b10, b20, sg0)

            @pl.when(i2 > 0)
            def _w0():
                wait_store(c0 - 2, pb0, so0)

            pack(a10, a20, b10, b20, pb0)
            store(c0, pb0, so0)

            fire(c0 + 2, a10, a20, b10, b20, sg0)
            wait_gather(c1, a11, a21, b11, b21, sg1)

            @pl.when(i2 > 0)
            def _w1():
                wait_store(c1 - 2, pb1, so1)

            pack(a11, a21, b11, b21, pb1)
            store(c1, pb1, so1)
            return carry

        lax.fori_loop(0, npair, pair, 0)

        c_last = nch - 1
        wait_gather(c_last, a10, a20, b10, b20, sg0)
        wait_store(c_last - 2, pb0, so0)
        pack(a10, a20, b10, b20, pb0)
        store(c_last, pb0, so0)
        wait_store(c_last - 1, pb1, so1)
        wait_store(c_last, pb0, so0)

    f = pl.kernel(
        body,
        out_type=jax.ShapeDtypeStruct((e // 2, hid), jnp.int32),
        mesh=mesh,
        scratch_types=[
            pltpu.VMEM((wrpt,), jnp.int32),
            pltpu.VMEM((wrpt,), jnp.int32),
            pltpu.VMEM((wrpt,), jnp.int32),
            pltpu.VMEM((wrpt,), jnp.int32),
            pltpu.VMEM((CW, hid), jnp.float32),
            pltpu.VMEM((CW, hid), jnp.float32),
            pltpu.VMEM((CW, hid), jnp.float32),
            pltpu.VMEM((CW, hid), jnp.float32),
            pltpu.VMEM((CW, hid), jnp.float32),
            pltpu.VMEM((CW, hid), jnp.float32),
            pltpu.VMEM((CW, hid), jnp.float32),
            pltpu.VMEM((CW, hid), jnp.float32),
            pltpu.VMEM((CW, hid), jnp.int32),
            pltpu.VMEM((CW, hid), jnp.int32),
            pltpu.SemaphoreType.DMA,
            pltpu.SemaphoreType.DMA,
            pltpu.SemaphoreType.DMA,
            pltpu.SemaphoreType.DMA,
            pltpu.SemaphoreType.DMA,
        ],
    )
    return f(g1, g2, row1d, col1d)


def _k3_body(s_ref, ea_ref, w1e_ref, b1_ref, w2_ref, b2_ref, o_ref):
    s2 = s_ref[...]
    beh = s2.shape[0]
    hid = s2.shape[1]
    w1e = w1e_ref[...]
    b1 = b1_ref[...]
    w2 = w2_ref[...]
    b2 = b2_ref[...]
    for half in range(2):
        sh = s2[:, half * (hid // 2):(half + 1) * (hid // 2)]
        lo = jax.lax.bitcast_convert_type(sh << 16, jnp.float32)
        hi = jax.lax.bitcast_convert_type(sh & -65536, jnp.float32)
        sf = jnp.concatenate([lo, hi], axis=1)
        rows = pl.ds(half * beh, beh)
        x = (sf
             + jnp.dot(ea_ref[rows, :], w1e, preferred_element_type=jnp.float32)
             + b1)
        x = _silu(x)
        y = jnp.dot(x, w2, preferred_element_type=jnp.float32) + b2
        o_ref[rows, :] = _silu(y)


def _k3(s, edge_attr, w1e, b1, w2, b2, be):
    e, ea = edge_attr.shape
    hid = s.shape[1]
    return pl.pallas_call(
        _k3_body,
        grid=(e // be,),
        in_specs=[
            pl.BlockSpec((be // 2, hid), lambda i: (i, 0)),
            pl.BlockSpec((be, ea), lambda i: (i, 0)),
            pl.BlockSpec((ea, hid), lambda i: (0, 0)),
            pl.BlockSpec((1, hid), lambda i: (0, 0)),
            pl.BlockSpec((hid, hid), lambda i: (0, 0)),
            pl.BlockSpec((1, hid), lambda i: (0, 0)),
        ],
        out_specs=pl.BlockSpec((be, hid), lambda i: (i, 0)),
        out_shape=jax.ShapeDtypeStruct((e, hid), jnp.float32),
    )(s, edge_attr, w1e, b1, w2, b2)


def _sc_segsum(mij, row3d, zeros_nh, n, e, hid, CH):
    epw = e // NW
    nch = epw // CH
    assert nch % 2 == 1 and nch >= 3
    npair = (nch - 1) // 2
    mesh = plsc.VectorSubcoreMesh(
        core_axis_name="c", subcore_axis_name="s",
        num_cores=NC, num_subcores=NS)

    def body(mij_hbm, row_hbm, z_hbm, out_hbm, idx, buf0, buf1, si0, si1, agg):
        cid = lax.axis_index("c")
        sid = lax.axis_index("s")
        wid = sid * NC + cid

        @pl.when(sid == 0)
        def _zero():
            pltpu.sync_copy(z_hbm, agg)

        plsc.subcore_barrier()

        pltpu.sync_copy(row_hbm.at[wid], idx)
        base = wid * epw

        def fire_in(c, buf, si):
            pltpu.async_copy(mij_hbm.at[pl.ds(base + c * CH, CH)], buf, si)

        def wait_in(c, buf, si):
            pltpu.make_async_copy(mij_hbm.at[pl.ds(base + c * CH, CH)], buf, si).wait()

        fire_in(0, buf0, si0)

        def pair(i2, c2):
            c0 = 2 * i2
            c1 = c0 + 1
            fire_in(c1, buf1, si1)
            wait_in(c0, buf0, si0)
            pltpu.sync_copy(buf0, agg.at[idx.at[c0]], add=True)
            fire_in(c0 + 2, buf0, si0)
            wait_in(c1, buf1, si1)
            pltpu.sync_copy(buf1, agg.at[idx.at[c1]], add=True)
            return c2

        lax.fori_loop(0, npair, pair, 0)

        c_last = nch - 1
        wait_in(c_last, buf0, si0)
        pltpu.sync_copy(buf0, agg.at[idx.at[c_last]], add=True)
        plsc.subcore_barrier()

        @pl.when(sid == 0)
        def _out():
            pltpu.sync_copy(agg, out_hbm.at[cid])

    f = pl.kernel(
        body,
        out_type=jax.ShapeDtypeStruct((NC, n, hid), jnp.float32),
        mesh=mesh,
        scratch_types=[
            pltpu.VMEM((nch, CH), jnp.int32),
            pltpu.VMEM((CH, hid), jnp.float32),
            pltpu.VMEM((CH, hid), jnp.float32),
            pltpu.SemaphoreType.DMA,
            pltpu.SemaphoreType.DMA,
            pltpu.VMEM_SHARED((n, hid), jnp.float32),
        ],
    )
    return f(mij, row3d, zeros_nh)


def _k5_body(h_ref, p_ref, w3h_ref, w3a_ref, b3_ref, w4_ref, b4_ref, o_ref):
    hb = h_ref[...]
    a = (p_ref[0] + p_ref[1]) * (1.0 / NORM)
    y = (jnp.dot(hb, w3h_ref[...], preferred_element_type=jnp.float32)
         + jnp.dot(a, w3a_ref[...], preferred_element_type=jnp.float32)
         + b3_ref[...])
    y = _silu(y)
    o_ref[...] = hb + jnp.dot(y, w4_ref[...], preferred_element_type=jnp.float32) + b4_ref[...]


def _k5(h, aggp, w3h, w3a, b3, w4, b4, bn):
    n, nf = h.shape
    hid = w3h.shape[1]
    return pl.pallas_call(
        _k5_body,
        grid=(n // bn,),
        in_specs=[
            pl.BlockSpec((bn, nf), lambda i: (i, 0)),
            pl.BlockSpec((NC, bn, hid), lambda i: (0, i, 0)),
            pl.BlockSpec((nf, hid), lambda i: (0, 0)),
            pl.BlockSpec((hid, hid), lambda i: (0, 0)),
            pl.BlockSpec((1, hid), lambda i: (0, 0)),
            pl.BlockSpec((hid, nf), lambda i: (0, 0)),
            pl.BlockSpec((1, nf), lambda i: (0, 0)),
        ],
        out_specs=pl.BlockSpec((bn, nf), lambda i: (i, 0)),
        out_shape=jax.ShapeDtypeStruct((n, nf), jnp.float32),
    )(h, aggp, w3h, w3a, b3, w4, b4)


def kernel(h, edge_index, edge_attr, W1, b1, W2, b2, W3, b3, W4, b4):
    n, nf = h.shape
    e, ea = edge_attr.shape
    hid = W2.shape[0]

    row = edge_index[0].astype(jnp.int32)
    col = edge_index[1].astype(jnp.int32)

    w1s = W1[:nf]
    w1t = W1[nf:2 * nf]
    w1e = W1[2 * nf:]
    b1r = b1.reshape(1, hid)
    w2p = W2
    b2r = b2.reshape(1, hid)
    zeros_nh = jnp.zeros((n, hid), jnp.float32)

    ch = 80
    nch = e // (NW * ch)
    row3d = row.reshape(NW, nch, ch)
    col3d = col.reshape(NW, nch, ch)

    be = 4000

    g1, g2 = _k1(h, w1s, w1t, 1000)
    s = _sc_gather_sum(g1, g2, row, col, e, hid, be // 2)
    mij = _k3(s, edge_attr, w1e, b1r, w2p, b2r, be)
    aggp = _sc_segsum(mij, row3d, zeros_nh, n, e, hid, ch)
    h_out = _k5(h, aggp, W3[:nf], W3[nf:], b3.reshape(1, hid), W4,
                b4.reshape(1, nf), 1000)
    return (h_out, mij)


# speedup vs baseline: 1.0255x; 1.0255x over previous
import functools

import jax
import jax.numpy as jnp
import numpy as np
from jax import lax
from jax.experimental import pallas as pl
from jax.experimental.pallas import tpu as pltpu
from jax.experimental.pallas import tpu_sc as plsc

NC = 2
NS = 16
NW = NC * NS

CH = 80
LANES = 16
NORM = 100.0


def _silu(x):
    return x * jax.nn.sigmoid(x)


def _k1_body(h_ref, w1s_ref, w1t_ref, o1_ref, o2_ref):
    hb = h_ref[...]
    o1_ref[...] = jnp.dot(hb, w1s_ref[...], preferred_element_type=jnp.float32)
    o2_ref[...] = jnp.dot(hb, w1t_ref[...], preferred_element_type=jnp.float32)


def _k1(h, w1s, w1t, bn):
    n, nf = h.shape
    hid = w1s.shape[1]
    return pl.pallas_call(
        _k1_body,
        grid=(n // bn,),
        in_specs=[
            pl.BlockSpec((bn, nf), lambda i: (i, 0)),
            pl.BlockSpec((nf, hid), lambda i: (0, 0)),
            pl.BlockSpec((nf, hid), lambda i: (0, 0)),
        ],
        out_specs=[
            pl.BlockSpec((bn, hid), lambda i: (i, 0)),
            pl.BlockSpec((bn, hid), lambda i: (i, 0)),
        ],
        out_shape=[
            jax.ShapeDtypeStruct((n, hid), jnp.float32),
            jax.ShapeDtypeStruct((n, hid), jnp.float32),
        ],
    )(h, w1s, w1t)


def _sc_gather_sum(g1, g2, row1d, col1d, e, hid, beh):
    wrpt = e // (2 * NW)
    CW = 40
    nch = wrpt // CW
    G = 1000
    ng = wrpt // G
    assert nch >= 4
    assert beh % G == 0 and wrpt % G == 0 and G % CW == 0
    npair = nch // 2
    mesh = plsc.VectorSubcoreMesh(
        core_axis_name="c", subcore_axis_name="s",
        num_cores=NC, num_subcores=NS)

    def body(g1_hbm, g2_hbm, row_hbm, col_hbm, out_hbm,
             iar, iac, ibr, ibc,
             a10, a20, b10, b20, a11, a21, b11, b21,
             pb0, pb1, sg0, sg1, so0, so1, sp):
        cid = lax.axis_index("c")
        sid = lax.axis_index("s")
        wid = sid * NC + cid
        m0t = wid * wrpt

        cps = []
        for g in range(ng):
            gg = wid * ng + g
            bi = gg >> 1
            mm = (gg & 1) * G
            src_a = pl.multiple_of(2 * beh * bi + mm, 8)
            src_b = src_a + beh
            dst = pl.ds(g * G, G)
            cps.append(pltpu.async_copy(row_hbm.at[pl.ds(src_a, G)], iar.at[dst], sp))
            cps.append(pltpu.async_copy(col_hbm.at[pl.ds(src_a, G)], iac.at[dst], sp))
            cps.append(pltpu.async_copy(row_hbm.at[pl.ds(src_b, G)], ibr.at[dst], sp))
            cps.append(pltpu.async_copy(col_hbm.at[pl.ds(src_b, G)], ibc.at[dst], sp))
        for cp in cps:
            cp.wait()

        def fire(c, a1, a2, b1_, b2_, sg):
            sl = pl.ds(c * CW, CW)
            pltpu.async_copy(g1_hbm.at[iar.at[sl]], a1, sg)
            pltpu.async_copy(g2_hbm.at[iac.at[sl]], a2, sg)
            pltpu.async_copy(g1_hbm.at[ibr.at[sl]], b1_, sg)
            pltpu.async_copy(g2_hbm.at[ibc.at[sl]], b2_, sg)

        def wait_gather(c, a1, a2, b1_, b2_, sg):
            sl = pl.ds(c * CW, CW)
            pltpu.make_async_copy(g1_hbm.at[iar.at[sl]], a1, sg).wait()
            pltpu.make_async_copy(g2_hbm.at[iac.at[sl]], a2, sg).wait()
            pltpu.make_async_copy(g1_hbm.at[ibr.at[sl]], b1_, sg).wait()
            pltpu.make_async_copy(g2_hbm.at[ibc.at[sl]], b2_, sg).wait()

        def bf16_word(x1, x2, r, sl_lo, sl_hi):
            va = x1[r, sl_lo] + x2[r, sl_lo]
            vb = x1[r, sl_hi] + x2[r, sl_hi]
            ia = lax.bitcast_convert_type(va, jnp.int32)
            ib = lax.bitcast_convert_type(vb, jnp.int32)
            ra = ia + 0x7FFF + ((ia >> 16) & 1)
            rb = ib + 0x7FFF + ((ib >> 16) & 1)
            return ((ra >> 16) & 0xFFFF) | (rb & -65536)

        def pack(a1, a2, b1_, b2_, pb):
            hw = hid // 2

            def rowfn(rr, c2):
                for t in range(hw // LANES):
                    sl_lo = pl.ds(t * LANES, LANES)
                    sl_hi = pl.ds(hw + t * LANES, LANES)
                    pb[rr, sl_lo] = bf16_word(a1, a2, rr, sl_lo, sl_hi)
                    pb[rr, sl_hi] = bf16_word(b1_, b2_, rr, sl_lo, sl_hi)
                return c2

            lax.fori_loop(0, CW, rowfn, 0)

        def store(c, pb, so):
            pltpu.async_copy(pb, out_hbm.at[pl.ds(m0t + c * CW, CW)], so)

        def wait_store(c, pb, so):
            pltpu.make_async_copy(
                pb, out_hbm.at[pl.ds(m0t + c * CW, CW)], so).wait()

        fire(0, a10, a20, b10, b20, sg0)

        def pair(i2, carry):
            c0 = 2 * i2
            c1 = c0 + 1
            fire(c1, a11, a21, b11, b21, sg1)
            wait_gather(c0, a10, a20, b10, b20, sg0)

            @pl.when(i2 > 0)
            def _w0():
                wait_store(c0 - 2, pb0, so0)

            pack(a10, a20, b10, b20, pb0)
            store(c0, pb0, so0)

            @pl.when(c0 + 2 < nch)
            def _f0():
                fire(c0 + 2, a10, a20, b10, b20, sg0)

            wait_gather(c1, a11, a21, b11, b21, sg1)

            @pl.when(i2 > 0)
            def _w1():
                wait_store(c1 - 2, pb1, so1)

            pack(a11, a21, b11, b21, pb1)
            store(c1, pb1, so1)
            return carry

        lax.fori_loop(0, npair, pair, 0)

        if nch % 2 == 1:
            c_last = nch - 1
            wait_gather(c_last, a10, a20, b10, b20, sg0)
            wait_store(c_last - 2, pb0, so0)
            pack(a10, a20, b10, b20, pb0)
            store(c_last, pb0, so0)
            wait_store(c_last - 1, pb1, so1)
            wait_store(c_last, pb0, so0)
        else:
            wait_store(nch - 2, pb0, so0)
            wait_store(nch - 1, pb1, so1)

    f = pl.kernel(
        body,
        out_type=jax.ShapeDtypeStruct((e // 2, hid), jnp.int32),
        mesh=mesh,
        scratch_types=[
            pltpu.VMEM((wrpt,), jnp.int32),
            pltpu.VMEM((wrpt,), jnp.int32),
            pltpu.VMEM((wrpt,), jnp.int32),
            pltpu.VMEM((wrpt,), jnp.int32),
            pltpu.VMEM((CW, hid), jnp.float32),
            pltpu.VMEM((CW, hid), jnp.float32),
            pltpu.VMEM((CW, hid), jnp.float32),
            pltpu.VMEM((CW, hid), jnp.float32),
            pltpu.VMEM((CW, hid), jnp.float32),
            pltpu.VMEM((CW, hid), jnp.float32),
            pltpu.VMEM((CW, hid), jnp.float32),
            pltpu.VMEM((CW, hid), jnp.float32),
            pltpu.VMEM((CW, hid), jnp.int32),
            pltpu.VMEM((CW, hid), jnp.int32),
            pltpu.SemaphoreType.DMA,
            pltpu.SemaphoreType.DMA,
            pltpu.SemaphoreType.DMA,
            pltpu.SemaphoreType.DMA,
            pltpu.SemaphoreType.DMA,
        ],
    )
    return f(g1, g2, row1d, col1d)


def _k3_body(s_ref, ea_ref, w1e_ref, b1_ref, w2_ref, b2_ref, o_ref):
    s2 = s_ref[...]
    beh = s2.shape[0]
    hid = s2.shape[1]
    w1e = w1e_ref[...]
    b1 = b1_ref[...]
    w2 = w2_ref[...]
    b2 = b2_ref[...]
    for half in range(2):
        sh = s2[:, half * (hid // 2):(half + 1) * (hid // 2)]
        lo = jax.lax.bitcast_convert_type(sh << 16, jnp.float32)
        hi = jax.lax.bitcast_convert_type(sh & -65536, jnp.float32)
        sf = jnp.concatenate([lo, hi], axis=1)
        rows = pl.ds(half * beh, beh)
        x = (sf
             + jnp.dot(ea_ref[rows, :], w1e, preferred_element_type=jnp.float32)
             + b1)
        x = _silu(x)
        y = jnp.dot(x, w2, preferred_element_type=jnp.float32) + b2
        o_ref[rows, :] = _silu(y)


def _k3_body_aliased(p_ref, s_ref, ea_ref, w1e_ref, b1_ref, w2_ref, b2_ref, o_ref):
    del p_ref
    _k3_body(s_ref, ea_ref, w1e_ref, b1_ref, w2_ref, b2_ref, o_ref)


def _k3_part(s, ea_part, w1e, b1, w2, b2, be, e_tot, blk0, partial):
    ep, ea = ea_part.shape
    hid = s.shape[1]
    specs = [
        pl.BlockSpec((be // 2, hid), lambda i: (i, 0)),
        pl.BlockSpec((be, ea), lambda i: (i, 0)),
        pl.BlockSpec((ea, hid), lambda i: (0, 0)),
        pl.BlockSpec((1, hid), lambda i: (0, 0)),
        pl.BlockSpec((hid, hid), lambda i: (0, 0)),
        pl.BlockSpec((1, hid), lambda i: (0, 0)),
    ]
    out_spec = pl.BlockSpec((be, hid), lambda i: (i + blk0, 0))
    out_shape = jax.ShapeDtypeStruct((e_tot, hid), jnp.float32)
    if partial is None:
        return pl.pallas_call(
            _k3_body,
            grid=(ep // be,),
            in_specs=specs,
            out_specs=out_spec,
            out_shape=out_shape,
        )(s, ea_part, w1e, b1, w2, b2)
    specs = [pl.BlockSpec((8, hid), lambda i: (0, 0))] + specs
    return pl.pallas_call(
        _k3_body_aliased,
        grid=(ep // be,),
        in_specs=specs,
        out_specs=out_spec,
        out_shape=out_shape,
        input_output_aliases={0: 0},
    )(partial, s, ea_part, w1e, b1, w2, b2)


def _sc_segsum(mij, row3d, zeros_nh, n, e, hid, CH):
    epw = e // NW
    nch = epw // CH
    assert nch % 2 == 1 and nch >= 3
    npair = (nch - 1) // 2
    mesh = plsc.VectorSubcoreMesh(
        core_axis_name="c", subcore_axis_name="s",
        num_cores=NC, num_subcores=NS)

    def body(mij_hbm, row_hbm, z_hbm, out_hbm, idx, buf0, buf1, si0, si1, agg):
        cid = lax.axis_index("c")
        sid = lax.axis_index("s")
        wid = sid * NC + cid

        @pl.when(sid == 0)
        def _zero():
            pltpu.sync_copy(z_hbm, agg)

        plsc.subcore_barrier()

        pltpu.sync_copy(row_hbm.at[wid], idx)
        base = wid * epw

        def fire_in(c, buf, si):
            pltpu.async_copy(mij_hbm.at[pl.ds(base + c * CH, CH)], buf, si)

        def wait_in(c, buf, si):
            pltpu.make_async_copy(mij_hbm.at[pl.ds(base + c * CH, CH)], buf, si).wait()

        fire_in(0, buf0, si0)

        def pair(i2, c2):
            c0 = 2 * i2
            c1 = c0 + 1
            fire_in(c1, buf1, si1)
            wait_in(c0, buf0, si0)
            pltpu.sync_copy(buf0, agg.at[idx.at[c0]], add=True)
            fire_in(c0 + 2, buf0, si0)
            wait_in(c1, buf1, si1)
            pltpu.sync_copy(buf1, agg.at[idx.at[c1]], add=True)
            return c2

        lax.fori_loop(0, npair, pair, 0)

        c_last = nch - 1
        wait_in(c_last, buf0, si0)
        pltpu.sync_copy(buf0, agg.at[idx.at[c_last]], add=True)
        plsc.subcore_barrier()

        @pl.when(sid == 0)
        def _out():
            pltpu.sync_copy(agg, out_hbm.at[cid])

    f = pl.kernel(
        body,
        out_type=jax.ShapeDtypeStruct((NC, n, hid), jnp.float32),
        mesh=mesh,
        scratch_types=[
            pltpu.VMEM((nch, CH), jnp.int32),
            pltpu.VMEM((CH, hid), jnp.float32),
            pltpu.VMEM((CH, hid), jnp.float32),
            pltpu.SemaphoreType.DMA,
            pltpu.SemaphoreType.DMA,
            pltpu.VMEM_SHARED((n, hid), jnp.float32),
        ],
    )
    return f(mij, row3d, zeros_nh)


def _k5_body(h_ref, p_ref, w3h_ref, w3a_ref, b3_ref, w4_ref, b4_ref, o_ref):
    hb = h_ref[...]
    a = (p_ref[0] + p_ref[1]) * (1.0 / NORM)
    y = (jnp.dot(hb, w3h_ref[...], preferred_element_type=jnp.float32)
         + jnp.dot(a, w3a_ref[...], preferred_element_type=jnp.float32)
         + b3_ref[...])
    y = _silu(y)
    o_ref[...] = hb + jnp.dot(y, w4_ref[...], preferred_element_type=jnp.float32) + b4_ref[...]


def _k5(h, aggp, w3h, w3a, b3, w4, b4, bn):
    n, nf = h.shape
    hid = w3h.shape[1]
    return pl.pallas_call(
        _k5_body,
        grid=(n // bn,),
        in_specs=[
            pl.BlockSpec((bn, nf), lambda i: (i, 0)),
            pl.BlockSpec((NC, bn, hid), lambda i: (0, i, 0)),
            pl.BlockSpec((nf, hid), lambda i: (0, 0)),
            pl.BlockSpec((hid, hid), lambda i: (0, 0)),
            pl.BlockSpec((1, hid), lambda i: (0, 0)),
            pl.BlockSpec((hid, nf), lambda i: (0, 0)),
            pl.BlockSpec((1, nf), lambda i: (0, 0)),
        ],
        out_specs=pl.BlockSpec((bn, nf), lambda i: (i, 0)),
        out_shape=jax.ShapeDtypeStruct((n, nf), jnp.float32),
    )(h, aggp, w3h, w3a, b3, w4, b4)


def kernel(h, edge_index, edge_attr, W1, b1, W2, b2, W3, b3, W4, b4):
    n, nf = h.shape
    e, ea = edge_attr.shape
    hid = W2.shape[0]

    row = edge_index[0].astype(jnp.int32)
    col = edge_index[1].astype(jnp.int32)

    w1s = W1[:nf]
    w1t = W1[nf:2 * nf]
    w1e = W1[2 * nf:]
    b1r = b1.reshape(1, hid)
    w2p = W2
    b2r = b2.reshape(1, hid)
    zeros_nh = jnp.zeros((n, hid), jnp.float32)

    ch = 80
    nch = e // (NW * ch)
    row3d = row.reshape(NW, nch, ch)
    col3d = col.reshape(NW, nch, ch)

    be = 4000
    e_a = 192000

    g1, g2 = _k1(h, w1s, w1t, 1000)
    s_a = _sc_gather_sum(g1, g2, row[:e_a], col[:e_a], e_a, hid, be // 2)
    s_b = _sc_gather_sum(g1, g2, row[e_a:], col[e_a:], e - e_a, hid, be // 2)
    mij_a = _k3_part(s_a, edge_attr[:e_a], w1e, b1r, w2p, b2r, be, e, 0, None)
    mij = _k3_part(s_b, edge_attr[e_a:], w1e, b1r, w2p, b2r, be, e,
                   e_a // be, mij_a)
    aggp = _sc_segsum(mij, row3d, zeros_nh, n, e, hid, ch)
    h_out = _k5(h, aggp, W3[:nf], W3[nf:], b3.reshape(1, hid), W4,
                b4.reshape(1, nf), 1000)
    return (h_out, mij)


# speedup vs baseline: 1.0268x; 1.0012x over previous
import jax
import jax.numpy as jnp
from jax import lax
from jax.experimental import pallas as pl
from jax.experimental.pallas import tpu as pltpu
from jax.experimental.pallas import tpu_sc as plsc

NC = 2
NS = 16
NW = NC * NS

CH = 80
LANES = 16
NORM = 100.0


def _silu(x):
    return x * jax.nn.sigmoid(x)


def _k1_body(h_ref, w1s_ref, w1t_ref, o1_ref, o2_ref):
    hb = h_ref[...]
    o1_ref[...] = jnp.dot(hb, w1s_ref[...], preferred_element_type=jnp.float32)
    o2_ref[...] = jnp.dot(hb, w1t_ref[...], preferred_element_type=jnp.float32)


def _k1(h, w1s, w1t, bn):
    n, nf = h.shape
    hid = w1s.shape[1]
    return pl.pallas_call(
        _k1_body,
        grid=(n // bn,),
        in_specs=[
            pl.BlockSpec((bn, nf), lambda i: (i, 0)),
            pl.BlockSpec((nf, hid), lambda i: (0, 0)),
            pl.BlockSpec((nf, hid), lambda i: (0, 0)),
        ],
        out_specs=[
            pl.BlockSpec((bn, hid), lambda i: (i, 0)),
            pl.BlockSpec((bn, hid), lambda i: (i, 0)),
        ],
        out_shape=[
            jax.ShapeDtypeStruct((n, hid), jnp.float32),
            jax.ShapeDtypeStruct((n, hid), jnp.float32),
        ],
    )(h, w1s, w1t)


def _sc_gather_sum(g1, g2, row1d, col1d, e, hid, beh):
    wrpt = e // (2 * NW)
    CW = 40
    nch = wrpt // CW
    G = 1000
    ng = wrpt // G
    assert nch >= 4
    assert beh % G == 0 and wrpt % G == 0 and G % CW == 0
    npair = nch // 2
    mesh = plsc.VectorSubcoreMesh(
        core_axis_name="c", subcore_axis_name="s",
        num_cores=NC, num_subcores=NS)

    def body(g1_hbm, g2_hbm, row_hbm, col_hbm, out_hbm,
             iar, iac, ibr, ibc,
             a10, a20, b10, b20, a11, a21, b11, b21,
             pb0, pb1, sg0, sg1, so0, so1, sp):
        cid = lax.axis_index("c")
        sid = lax.axis_index("s")
        wid = sid * NC + cid
        m0t = wid * wrpt

        cps = []
        for g in range(ng):
            gg = wid * ng + g
            bi = gg >> 1
            mm = (gg & 1) * G
            src_a = pl.multiple_of(2 * beh * bi + mm, 8)
            src_b = src_a + beh
            dst = pl.ds(g * G, G)
            cps.append(pltpu.async_copy(row_hbm.at[pl.ds(src_a, G)], iar.at[dst], sp))
            cps.append(pltpu.async_copy(col_hbm.at[pl.ds(src_a, G)], iac.at[dst], sp))
            cps.append(pltpu.async_copy(row_hbm.at[pl.ds(src_b, G)], ibr.at[dst], sp))
            cps.append(pltpu.async_copy(col_hbm.at[pl.ds(src_b, G)], ibc.at[dst], sp))
        for cp in cps:
            cp.wait()

        def fire(c, a1, a2, b1_, b2_, sg):
            sl = pl.ds(c * CW, CW)
            pltpu.async_copy(g1_hbm.at[iar.at[sl]], a1, sg)
            pltpu.async_copy(g2_hbm.at[iac.at[sl]], a2, sg)
            pltpu.async_copy(g1_hbm.at[ibr.at[sl]], b1_, sg)
            pltpu.async_copy(g2_hbm.at[ibc.at[sl]], b2_, sg)

        def wait_gather(c, a1, a2, b1_, b2_, sg):
            sl = pl.ds(c * CW, CW)
            pltpu.make_async_copy(g1_hbm.at[iar.at[sl]], a1, sg).wait()
            pltpu.make_async_copy(g2_hbm.at[iac.at[sl]], a2, sg).wait()
            pltpu.make_async_copy(g1_hbm.at[ibr.at[sl]], b1_, sg).wait()
            pltpu.make_async_copy(g2_hbm.at[ibc.at[sl]], b2_, sg).wait()

        def bf16_word(x1, x2, r, sl_lo, sl_hi):
            va = x1[r, sl_lo] + x2[r, sl_lo]
            vb = x1[r, sl_hi] + x2[r, sl_hi]
            ia = lax.bitcast_convert_type(va, jnp.int32)
            ib = lax.bitcast_convert_type(vb, jnp.int32)
            ra = ia + 0x7FFF + ((ia >> 16) & 1)
            rb = ib + 0x7FFF + ((ib >> 16) & 1)
            return ((ra >> 16) & 0xFFFF) | (rb & -65536)

        def pack(a1, a2, b1_, b2_, pb):
            hw = hid // 2

            def rowfn(rr, c2):
                for t in range(hw // LANES):
                    sl_lo = pl.ds(t * LANES, LANES)
                    sl_hi = pl.ds(hw + t * LANES, LANES)
                    pb[rr, sl_lo] = bf16_word(a1, a2, rr, sl_lo, sl_hi)
                    pb[rr, sl_hi] = bf16_word(b1_, b2_, rr, sl_lo, sl_hi)
                return c2

            lax.fori_loop(0, CW, rowfn, 0)

        def store(c, pb, so):
            pltpu.async_copy(pb, out_hbm.at[pl.ds(m0t + c * CW, CW)], so)

        def wait_store(c, pb, so):
            pltpu.make_async_copy(
                pb, out_hbm.at[pl.ds(m0t + c * CW, CW)], so).wait()

        fire(0, a10, a20, b10, b20, sg0)

        def pair(i2, carry):
            c0 = 2 * i2
            c1 = c0 + 1
            fire(c1, a11, a21, b11, b21, sg1)
            wait_gather(c0, a10, a20, b10, b20, sg0)

            @pl.when(i2 > 0)
            def _w0():
                wait_store(c0 - 2, pb0, so0)

            pack(a10, a20, b10, b20, pb0)
            store(c0, pb0, so0)

            @pl.when(c0 + 2 < nch)
            def _f0():
                fire(c0 + 2, a10, a20, b10, b20, sg0)

            wait_gather(c1, a11, a21, b11, b21, sg1)

            @pl.when(i2 > 0)
            def _w1():
                wait_store(c1 - 2, pb1, so1)

            pack(a11, a21, b11, b21, pb1)
            store(c1, pb1, so1)
            return carry

        lax.fori_loop(0, npair, pair, 0)

        if nch % 2 == 1:
            c_last = nch - 1
            wait_gather(c_last, a10, a20, b10, b20, sg0)
            wait_store(c_last - 2, pb0, so0)
            pack(a10, a20, b10, b20, pb0)
            store(c_last, pb0, so0)
            wait_store(c_last - 1, pb1, so1)
            wait_store(c_last, pb0, so0)
        else:
            wait_store(nch - 2, pb0, so0)
            wait_store(nch - 1, pb1, so1)

    f = pl.kernel(
        body,
        out_type=jax.ShapeDtypeStruct((e // 2, hid), jnp.int32),
        mesh=mesh,
        scratch_types=[
            pltpu.VMEM((wrpt,), jnp.int32),
            pltpu.VMEM((wrpt,), jnp.int32),
            pltpu.VMEM((wrpt,), jnp.int32),
            pltpu.VMEM((wrpt,), jnp.int32),
            pltpu.VMEM((CW, hid), jnp.float32),
            pltpu.VMEM((CW, hid), jnp.float32),
            pltpu.VMEM((CW, hid), jnp.float32),
            pltpu.VMEM((CW, hid), jnp.float32),
            pltpu.VMEM((CW, hid), jnp.float32),
            pltpu.VMEM((CW, hid), jnp.float32),
            pltpu.VMEM((CW, hid), jnp.float32),
            pltpu.VMEM((CW, hid), jnp.float32),
            pltpu.VMEM((CW, hid), jnp.int32),
            pltpu.VMEM((CW, hid), jnp.int32),
            pltpu.SemaphoreType.DMA,
            pltpu.SemaphoreType.DMA,
            pltpu.SemaphoreType.DMA,
            pltpu.SemaphoreType.DMA,
            pltpu.SemaphoreType.DMA,
        ],
    )
    return f(g1, g2, row1d, col1d)


def _k3_body(s_ref, ea_ref, w1e_ref, b1_ref, w2_ref, b2_ref, o_ref):
    s2 = s_ref[...]
    beh = s2.shape[0]
    hid = s2.shape[1]
    w1e = w1e_ref[...]
    b1 = b1_ref[...]
    w2 = w2_ref[...]
    b2 = b2_ref[...]
    for half in range(2):
        sh = s2[:, half * (hid // 2):(half + 1) * (hid // 2)]
        lo = jax.lax.bitcast_convert_type(sh << 16, jnp.float32)
        hi = jax.lax.bitcast_convert_type(sh & -65536, jnp.float32)
        sf = jnp.concatenate([lo, hi], axis=1)
        rows = pl.ds(half * beh, beh)
        x = (sf
             + jnp.dot(ea_ref[rows, :], w1e, preferred_element_type=jnp.float32)
             + b1)
        x = _silu(x)
        y = jnp.dot(x, w2, preferred_element_type=jnp.float32) + b2
        o_ref[rows, :] = _silu(y)


def _k3_body_aliased(p_ref, s_ref, ea_ref, w1e_ref, b1_ref, w2_ref, b2_ref, o_ref):
    del p_ref
    _k3_body(s_ref, ea_ref, w1e_ref, b1_ref, w2_ref, b2_ref, o_ref)


def _k3_part(s, ea_part, w1e, b1, w2, b2, be, e_tot, blk0, partial):
    ep, ea = ea_part.shape
    hid = s.shape[1]
    specs = [
        pl.BlockSpec((be // 2, hid), lambda i: (i, 0)),
        pl.BlockSpec((be, ea), lambda i: (i, 0)),
        pl.BlockSpec((ea, hid), lambda i: (0, 0)),
        pl.BlockSpec((1, hid), lambda i: (0, 0)),
        pl.BlockSpec((hid, hid), lambda i: (0, 0)),
        pl.BlockSpec((1, hid), lambda i: (0, 0)),
    ]
    out_spec = pl.BlockSpec((be, hid), lambda i: (i + blk0, 0))
    out_shape = jax.ShapeDtypeStruct((e_tot, hid), jnp.float32)
    if partial is None:
        return pl.pallas_call(
            _k3_body,
            grid=(ep // be,),
            in_specs=specs,
            out_specs=out_spec,
            out_shape=out_shape,
        )(s, ea_part, w1e, b1, w2, b2)
    specs = [pl.BlockSpec((8, hid), lambda i: (0, 0))] + specs
    return pl.pallas_call(
        _k3_body_aliased,
        grid=(ep // be,),
        in_specs=specs,
        out_specs=out_spec,
        out_shape=out_shape,
        input_output_aliases={0: 0},
    )(partial, s, ea_part, w1e, b1, w2, b2)


def _sc_segsum(mij, row3d, zeros_nh, n, e, hid, CH):
    epw = e // NW
    nch = epw // CH
    assert nch % 2 == 1 and nch >= 3
    npair = (nch - 1) // 2
    mesh = plsc.VectorSubcoreMesh(
        core_axis_name="c", subcore_axis_name="s",
        num_cores=NC, num_subcores=NS)

    def body(mij_hbm, row_hbm, z_hbm, out_hbm, idx, buf0, buf1, si0, si1, agg):
        cid = lax.axis_index("c")
        sid = lax.axis_index("s")
        wid = sid * NC + cid

        @pl.when(sid == 0)
        def _zero():
            pltpu.sync_copy(z_hbm, agg)

        plsc.subcore_barrier()

        pltpu.sync_copy(row_hbm.at[wid], idx)
        base = wid * epw

        def fire_in(c, buf, si):
            pltpu.async_copy(mij_hbm.at[pl.ds(base + c * CH, CH)], buf, si)

        def wait_in(c, buf, si):
            pltpu.make_async_copy(mij_hbm.at[pl.ds(base + c * CH, CH)], buf, si).wait()

        fire_in(0, buf0, si0)

        def pair(i2, c2):
            c0 = 2 * i2
            c1 = c0 + 1
            fire_in(c1, buf1, si1)
            wait_in(c0, buf0, si0)
            pltpu.sync_copy(buf0, agg.at[idx.at[c0]], add=True)
            fire_in(c0 + 2, buf0, si0)
            wait_in(c1, buf1, si1)
            pltpu.sync_copy(buf1, agg.at[idx.at[c1]], add=True)
            return c2

        lax.fori_loop(0, npair, pair, 0)

        c_last = nch - 1
        wait_in(c_last, buf0, si0)
        pltpu.sync_copy(buf0, agg.at[idx.at[c_last]], add=True)
        plsc.subcore_barrier()

        @pl.when(sid == 0)
        def _out():
            pltpu.sync_copy(agg, out_hbm.at[cid])

    f = pl.kernel(
        body,
        out_type=jax.ShapeDtypeStruct((NC, n, hid), jnp.float32),
        mesh=mesh,
        scratch_types=[
            pltpu.VMEM((nch, CH), jnp.int32),
            pltpu.VMEM((CH, hid), jnp.float32),
            pltpu.VMEM((CH, hid), jnp.float32),
            pltpu.SemaphoreType.DMA,
            pltpu.SemaphoreType.DMA,
            pltpu.VMEM_SHARED((n, hid), jnp.float32),
        ],
    )
    return f(mij, row3d, zeros_nh)


def _k5_body(h_ref, p_ref, w3h_ref, w3a_ref, b3_ref, w4_ref, b4_ref, o_ref):
    hb = h_ref[...]
    a = (p_ref[0] + p_ref[1]) * (1.0 / NORM)
    y = (jnp.dot(hb, w3h_ref[...], preferred_element_type=jnp.float32)
         + jnp.dot(a, w3a_ref[...], preferred_element_type=jnp.float32)
         + b3_ref[...])
    y = _silu(y)
    o_ref[...] = hb + jnp.dot(y, w4_ref[...], preferred_element_type=jnp.float32) + b4_ref[...]


def _k5(h, aggp, w3h, w3a, b3, w4, b4, bn):
    n, nf = h.shape
    hid = w3h.shape[1]
    return pl.pallas_call(
        _k5_body,
        grid=(n // bn,),
        in_specs=[
            pl.BlockSpec((bn, nf), lambda i: (i, 0)),
            pl.BlockSpec((NC, bn, hid), lambda i: (0, i, 0)),
            pl.BlockSpec((nf, hid), lambda i: (0, 0)),
            pl.BlockSpec((hid, hid), lambda i: (0, 0)),
            pl.BlockSpec((1, hid), lambda i: (0, 0)),
            pl.BlockSpec((hid, nf), lambda i: (0, 0)),
            pl.BlockSpec((1, nf), lambda i: (0, 0)),
        ],
        out_specs=pl.BlockSpec((bn, nf), lambda i: (i, 0)),
        out_shape=jax.ShapeDtypeStruct((n, nf), jnp.float32),
    )(h, aggp, w3h, w3a, b3, w4, b4)


def kernel(h, edge_index, edge_attr, W1, b1, W2, b2, W3, b3, W4, b4):
    n, nf = h.shape
    e, ea = edge_attr.shape
    hid = W2.shape[0]

    row = edge_index[0].astype(jnp.int32)
    col = edge_index[1].astype(jnp.int32)

    w1s = W1[:nf]
    w1t = W1[nf:2 * nf]
    w1e = W1[2 * nf:]
    b1r = b1.reshape(1, hid)
    w2p = W2
    b2r = b2.reshape(1, hid)
    zeros_nh = jnp.zeros((n, hid), jnp.float32)

    ch = 80
    nch = e // (NW * ch)
    row3d = row.reshape(NW, nch, ch)
    col3d = col.reshape(NW, nch, ch)

    be = 4000
    e_a = 192000

    g1, g2 = _k1(h, w1s, w1t, 1000)
    s_a = _sc_gather_sum(g1, g2, row[:e_a], col[:e_a], e_a, hid, be // 2)
    s_b = _sc_gather_sum(g1, g2, row[e_a:], col[e_a:], e - e_a, hid, be // 2)
    mij_a = _k3_part(s_a, edge_attr[:e_a], w1e, b1r, w2p, b2r, be, e, 0, None)
    mij = _k3_part(s_b, edge_attr[e_a:], w1e, b1r, w2p, b2r, be, e,
                   e_a // be, mij_a)
    aggp = _sc_segsum(mij, row3d, zeros_nh, n, e, hid, ch)
    h_out = _k5(h, aggp, W3[:nf], W3[nf:], b3.reshape(1, hid), W4,
                b4.reshape(1, nf), 1000)
    return (h_out, mij)
